# baseline (speedup 1.0000x reference)
import jax
import jax.numpy as jnp
from jax import lax
from jax.experimental import pallas as pl
from jax.experimental.pallas import tpu as pltpu
from jax.experimental.pallas import tpu_sc as plsc

B = 2
T = 2048
D = 1024
E = 8
K = 2
N = B * T
S = N * K

TBLK = 512
NT = N // TBLK

BLK = 256
S_PAD = S + E * BLK
NB = S_PAD // BLK

NC = 2
NS = 16
NW = NC * NS


def _gate_kernel(logits_ref, v_ref, pos_ref, cnt_ref, p_s, carry_s):
    ph = pl.program_id(0)
    t = pl.program_id(1)

    logits = logits_ref[...]
    mx = jnp.max(logits, axis=-1, keepdims=True)
    pexp = jnp.exp(logits - mx)
    probs = pexp / jnp.sum(pexp, axis=-1, keepdims=True)
    iota = lax.broadcasted_iota(jnp.int32, (TBLK, E), 1)
    m1 = jnp.max(probs, axis=-1, keepdims=True)
    i1 = jnp.min(jnp.where(probs == m1, iota, E), axis=-1, keepdims=True)
    p2 = jnp.where(iota == i1, -1.0, probs)
    m2 = jnp.max(p2, axis=-1, keepdims=True)
    i2 = jnp.min(jnp.where(p2 == m2, iota, E), axis=-1, keepdims=True)
    v_ref[...] = jnp.concatenate([m1, m2], axis=-1)

    ohsum = ((iota == i1) | (iota == i2)).astype(jnp.float32)

    @pl.when(ph == 0)
    def _phase_count():
        @pl.when(t == 0)
        def _init():
            carry_s[...] = jnp.zeros_like(carry_s)

        r_iota = lax.broadcasted_iota(jnp.int32, (TBLK, TBLK), 0)
        c_iota = lax.broadcasted_iota(jnp.int32, (TBLK, TBLK), 1)
        tril = (c_iota < r_iota).astype(jnp.float32)
        pref = jnp.dot(tril, ohsum, preferred_element_type=jnp.float32,
                       precision=lax.Precision.HIGHEST)
        p_s[pl.ds(t * TBLK, TBLK), :] = pref + carry_s[...]
        carry_s[...] = carry_s[...] + jnp.sum(ohsum, axis=0, keepdims=True)

    @pl.when(ph == 1)
    def _phase_pos():
        counts = carry_s[...]
        counts_pad = jnp.ceil(counts * (1.0 / BLK)) * BLK
        e_r = lax.broadcasted_iota(jnp.int32, (E, E), 0)
        e_c = lax.broadcasted_iota(jnp.int32, (E, E), 1)
        upper = (e_r < e_c).astype(jnp.float32)
        gstart = jnp.dot(counts_pad, upper,
                         preferred_element_type=jnp.float32,
                         precision=lax.Precision.HIGHEST)
        dest = gstart + p_s[pl.ds(t * TBLK, TBLK), :]
        pos0 = jnp.sum(jnp.where(iota == i1, dest, 0.0), axis=-1,
                       keepdims=True)
        pos1 = jnp.sum(jnp.where(iota == i2, dest, 0.0), axis=-1,
                       keepdims=True)
        pos_ref[...] = jnp.concatenate([pos0, pos1], axis=-1).astype(jnp.int32)
        cnt_ref[...] = counts.astype(jnp.int32)


def _gate(logits):
    return pl.pallas_call(
        _gate_kernel,
        grid=(2, NT),
        in_specs=[pl.BlockSpec((TBLK, E), lambda p, t: (t, 0))],
        out_specs=[pl.BlockSpec((TBLK, K), lambda p, t: (t, 0)),
                   pl.BlockSpec((TBLK, K), lambda p, t: (t, 0)),
                   pl.BlockSpec((1, E), lambda p, t: (0, 0))],
        out_shape=[jax.ShapeDtypeStruct((N, K), jnp.float32),
                   jax.ShapeDtypeStruct((N, K), jnp.int32),
                   jax.ShapeDtypeStruct((1, E), jnp.int32)],
        scratch_shapes=[pltpu.VMEM((N, E), jnp.float32),
                        pltpu.VMEM((1, E), jnp.float32)],
    )(logits)


_RCH = 32
_RPW = N // NW
_RNCH = _RPW // _RCH


def _sc_reorder_body(x_hbm, idx0_hbm, idx1_hbm, out_hbm,
                     i0_v, i1_v, buf0, buf1, ls0, ls1, ss0, ss1):
    wid = lax.axis_index("s") * NC + lax.axis_index("c")
    base = wid * _RPW
    pltpu.sync_copy(idx0_hbm.at[wid], i0_v)
    pltpu.sync_copy(idx1_hbm.at[wid], i1_v)
    bufs = (buf0, buf1)
    lsems = (ls0, ls1)
    ssems = (ss0, ss1)
    pend_l = {}
    pend_s = {}

    def start_load(c):
        pend_l[c] = pltpu.async_copy(
            x_hbm.at[pl.ds(base + c * _RCH, _RCH)], bufs[c % 2],
            lsems[c % 2])

    start_load(0)
    for c in range(_RNCH):
        if c + 1 < _RNCH:
            if c >= 1:
                pend_s[2 * (c - 1)].wait()
                pend_s[2 * (c - 1) + 1].wait()
            start_load(c + 1)
        pend_l[c].wait()
        pend_s[2 * c] = pltpu.async_copy(
            bufs[c % 2], out_hbm.at[i0_v.at[c]], ssems[c % 2])
        pend_s[2 * c + 1] = pltpu.async_copy(
            bufs[c % 2], out_hbm.at[i1_v.at[c]], ssems[c % 2])
    pend_s[2 * (_RNCH - 1)].wait()
    pend_s[2 * (_RNCH - 1) + 1].wait()
    if _RNCH >= 2:
        pend_s[2 * (_RNCH - 2)].wait()
        pend_s[2 * (_RNCH - 2) + 1].wait()


def _sc_reorder(x2d, idx0_3d, idx1_3d):
    ncol = x2d.shape[1]
    k = pl.kernel(
        _sc_reorder_body,
        mesh=plsc.VectorSubcoreMesh(core_axis_name="c", subcore_axis_name="s"),
        out_type=jax.ShapeDtypeStruct((S_PAD, ncol), x2d.dtype),
        scratch_types=[pltpu.VMEM((_RNCH, _RCH), jnp.int32),
                       pltpu.VMEM((_RNCH, _RCH), jnp.int32),
                       pltpu.VMEM((_RCH, ncol), x2d.dtype),
                       pltpu.VMEM((_RCH, ncol), x2d.dtype),
                       pltpu.SemaphoreType.DMA,
                       pltpu.SemaphoreType.DMA,
                       pltpu.SemaphoreType.DMA,
                       pltpu.SemaphoreType.DMA],
    )
    return k(x2d, idx0_3d, idx1_3d)


_GCH = 32


def _make_sc_gather_body(nr):
    rpw = nr // NW
    nch = rpw // _GCH

    def body(src_hbm, idx_hbm, out_hbm, idx_v, buf0, buf1, gs0, gs1, ws0, ws1):
        wid = lax.axis_index("s") * NC + lax.axis_index("c")
        base = wid * rpw
        pltpu.sync_copy(idx_hbm.at[pl.ds(base, rpw)], idx_v)
        bufs = (buf0, buf1)
        gsems = (gs0, gs1)
        wsems = (ws0, ws1)
        pend_g = {}
        pend_w = {}

        def start_gather(c):
            pend_g[c] = pltpu.async_copy(
                src_hbm.at[idx_v.at[pl.ds(c * _GCH, _GCH)]],
                bufs[c % 2], gsems[c % 2])

        start_gather(0)
        for c in range(nch):
            if c + 1 < nch:
                if c >= 1:
                    pend_w[c - 1].wait()
                start_gather(c + 1)
            pend_g[c].wait()
            pend_w[c] = pltpu.async_copy(
                bufs[c % 2], out_hbm.at[pl.ds(base + c * _GCH, _GCH)],
                wsems[c % 2])
        pend_w[nch - 1].wait()
        if nch >= 2:
            pend_w[nch - 2].wait()

    return body, rpw


def _sc_gather(src, idx, nr):
    body, rpw = _make_sc_gather_body(nr)
    k = pl.kernel(
        body,
        mesh=plsc.VectorSubcoreMesh(core_axis_name="c", subcore_axis_name="s"),
        out_type=jax.ShapeDtypeStruct((nr, D), src.dtype),
        scratch_types=[pltpu.VMEM((rpw,), jnp.int32),
                       pltpu.VMEM((_GCH, D), src.dtype),
                       pltpu.VMEM((_GCH, D), src.dtype),
                       pltpu.SemaphoreType.DMA,
                       pltpu.SemaphoreType.DMA,
                       pltpu.SemaphoreType.DMA,
                       pltpu.SemaphoreType.DMA],
    )
    return k(src, idx)


def _mm_kernel(be_ref, xs_ref, w_ref, eb_ref, os_ref, wbf_s):
    b = pl.program_id(0)

    @pl.when((b == 0) | (be_ref[b] != be_ref[jnp.maximum(b - 1, 0)]))
    def _cast_w():
        wbf_s[...] = w_ref[0].astype(jnp.bfloat16)

    y = jnp.dot(xs_ref[...], wbf_s[...], preferred_element_type=jnp.float32)
    os_ref[...] = y + eb_ref[0]


def _grouped_mm(xs, expert_w, eb3d, block_expert):
    grid_spec = pltpu.PrefetchScalarGridSpec(
        num_scalar_prefetch=1,
        grid=(NB,),
        in_specs=[
            pl.BlockSpec((BLK, D), lambda b, be: (b, 0)),
            pl.BlockSpec((1, D, D), lambda b, be: (be[b], 0, 0)),
            pl.BlockSpec((1, 1, D), lambda b, be: (be[b], 0, 0)),
        ],
        out_specs=pl.BlockSpec((BLK, D), lambda b, be: (b, 0)),
        scratch_shapes=[pltpu.VMEM((D, D), jnp.bfloat16)],
    )
    return pl.pallas_call(
        _mm_kernel,
        grid_spec=grid_spec,
        out_shape=jax.ShapeDtypeStruct((S_PAD, D), jnp.float32),
    )(block_expert, xs, expert_w, eb3d)


def _wadd_kernel(a_ref, b_ref, v_ref, o_ref):
    w = v_ref[...]
    o_ref[...] = a_ref[...] * w[:, 0:1] + b_ref[...] * w[:, 1:2]


def _tc_wadd(gcat, top2v):
    return pl.pallas_call(
        _wadd_kernel,
        grid=(NT,),
        in_specs=[pl.BlockSpec((TBLK, D), lambda t: (t, 0)),
                  pl.BlockSpec((TBLK, D), lambda t: (t + NT, 0)),
                  pl.BlockSpec((TBLK, K), lambda t: (t, 0))],
        out_specs=pl.BlockSpec((TBLK, D), lambda t: (t, 0)),
        out_shape=jax.ShapeDtypeStruct((N, D), jnp.float32),
    )(gcat, gcat, top2v)


@jax.jit
def kernel(hidden_states, gate_w, gate_b, expert_w, expert_b):
    x2d = hidden_states.reshape(N, D)
    eb3d = expert_b.reshape(E, 1, D)

    logits = (hidden_states @ gate_w + gate_b).reshape(N, E)

    top2v, pos, counts = _gate(logits)

    counts_pad = ((counts[0] + BLK - 1) // BLK) * BLK
    ends = jnp.cumsum(counts_pad)
    block_expert = jnp.minimum(
        jnp.sum((jnp.arange(NB, dtype=jnp.int32)[:, None] * BLK
                 >= ends[None, :]).astype(jnp.int32), 1),
        E - 1).astype(jnp.int32)
    idx0_3d = pos[:, 0].reshape(NW, _RNCH, _RCH)
    idx1_3d = pos[:, 1].reshape(NW, _RNCH, _RCH)
    idxcat = jnp.concatenate([pos[:, 0], pos[:, 1]]).astype(jnp.int32)

    x_bfi = lax.bitcast_convert_type(
        x2d.astype(jnp.bfloat16).reshape(N, D // 2, 2), jnp.int32)
    xs_i = _sc_reorder(x_bfi, idx0_3d, idx1_3d)
    xs = lax.bitcast_convert_type(xs_i, jnp.bfloat16).reshape(S_PAD, D)

    os_ = _grouped_mm(xs, expert_w, eb3d, block_expert)

    gcat = _sc_gather(os_, idxcat, 2 * N)
    out = _tc_wadd(gcat, top2v)

    return out.reshape(B, T, D)


# speedup vs baseline: 2.5458x; 2.5458x over previous
import jax
import jax.numpy as jnp
from jax import lax
from jax.experimental import pallas as pl
from jax.experimental.pallas import tpu as pltpu
from jax.experimental.pallas import tpu_sc as plsc

B = 2
T = 2048
D = 1024
E = 8
K = 2
N = B * T
S = N * K

TBLK = 1024
NT = N // TBLK

BLK = 128
S_PAD = S + E * BLK
NB = S_PAD // BLK

NC = 2
NS = 16
NW = NC * NS


def _gate_kernel(logits_ref, v_ref, pos_ref, cnt_ref, p_s, carry_s):
    ph = pl.program_id(0)
    t = pl.program_id(1)

    logits = logits_ref[...]
    mx = jnp.max(logits, axis=-1, keepdims=True)
    pexp = jnp.exp(logits - mx)
    probs = pexp / jnp.sum(pexp, axis=-1, keepdims=True)
    iota = lax.broadcasted_iota(jnp.int32, (TBLK, E), 1)
    m1 = jnp.max(probs, axis=-1, keepdims=True)
    i1 = jnp.min(jnp.where(probs == m1, iota, E), axis=-1, keepdims=True)
    p2 = jnp.where(iota == i1, -1.0, probs)
    m2 = jnp.max(p2, axis=-1, keepdims=True)
    i2 = jnp.min(jnp.where(p2 == m2, iota, E), axis=-1, keepdims=True)
    v_ref[...] = jnp.concatenate([m1, m2], axis=-1)

    ohsum = ((iota == i1) | (iota == i2)).astype(jnp.float32)

    @pl.when(ph == 0)
    def _phase_count():
        @pl.when(t == 0)
        def _init():
            carry_s[...] = jnp.zeros_like(carry_s)

        r_iota = lax.broadcasted_iota(jnp.int32, (TBLK, TBLK), 0)
        c_iota = lax.broadcasted_iota(jnp.int32, (TBLK, TBLK), 1)
        tril = (c_iota < r_iota).astype(jnp.float32)
        pref = jnp.dot(tril, ohsum, preferred_element_type=jnp.float32)
        p_s[pl.ds(t * TBLK, TBLK), :] = pref + carry_s[...]
        carry_s[...] = carry_s[...] + jnp.sum(ohsum, axis=0, keepdims=True)

    @pl.when(ph == 1)
    def _phase_pos():
        counts = carry_s[...]
        counts_pad = jnp.ceil(counts * (1.0 / BLK)) * BLK
        e_r = lax.broadcasted_iota(jnp.int32, (E, E), 0)
        e_c = lax.broadcasted_iota(jnp.int32, (E, E), 1)
        upper = (e_r < e_c).astype(jnp.float32)
        gstart = jnp.dot(counts_pad, upper,
                         preferred_element_type=jnp.float32,
                         precision=lax.Precision.HIGHEST)
        dest = gstart + p_s[pl.ds(t * TBLK, TBLK), :]
        pos0 = jnp.sum(jnp.where(iota == i1, dest, 0.0), axis=-1,
                       keepdims=True)
        pos1 = jnp.sum(jnp.where(iota == i2, dest, 0.0), axis=-1,
                       keepdims=True)
        pos_ref[...] = jnp.concatenate([pos0, pos1], axis=-1).astype(jnp.int32)
        cnt_ref[...] = counts.astype(jnp.int32)


def _gate(logits):
    return pl.pallas_call(
        _gate_kernel,
        grid=(2, NT),
        in_specs=[pl.BlockSpec((TBLK, E), lambda p, t: (t, 0))],
        out_specs=[pl.BlockSpec((TBLK, K), lambda p, t: (t, 0)),
                   pl.BlockSpec((TBLK, K), lambda p, t: (t, 0)),
                   pl.BlockSpec((1, E), lambda p, t: (0, 0))],
        out_shape=[jax.ShapeDtypeStruct((N, K), jnp.float32),
                   jax.ShapeDtypeStruct((N, K), jnp.int32),
                   jax.ShapeDtypeStruct((1, E), jnp.int32)],
        scratch_shapes=[pltpu.VMEM((N, E), jnp.float32),
                        pltpu.VMEM((1, E), jnp.float32)],
    )(logits)


_RCH = 32
_RPW = N // NW
_RNCH = _RPW // _RCH


def _sc_reorder_body(x_hbm, idx0_hbm, idx1_hbm, out_hbm,
                     i0_v, i1_v, buf0, buf1, ls0, ls1, ss0, ss1):
    wid = lax.axis_index("s") * NC + lax.axis_index("c")
    base = wid * _RPW
    pltpu.sync_copy(idx0_hbm.at[wid], i0_v)
    pltpu.sync_copy(idx1_hbm.at[wid], i1_v)
    bufs = (buf0, buf1)
    lsems = (ls0, ls1)
    ssems = (ss0, ss1)
    pend_l = {}
    pend_s = {}

    def start_load(c):
        pend_l[c] = pltpu.async_copy(
            x_hbm.at[pl.ds(base + c * _RCH, _RCH)], bufs[c % 2],
            lsems[c % 2])

    start_load(0)
    for c in range(_RNCH):
        if c + 1 < _RNCH:
            if c >= 1:
                pend_s[2 * (c - 1)].wait()
                pend_s[2 * (c - 1) + 1].wait()
            start_load(c + 1)
        pend_l[c].wait()
        pend_s[2 * c] = pltpu.async_copy(
            bufs[c % 2], out_hbm.at[i0_v.at[c]], ssems[c % 2])
        pend_s[2 * c + 1] = pltpu.async_copy(
            bufs[c % 2], out_hbm.at[i1_v.at[c]], ssems[c % 2])
    pend_s[2 * (_RNCH - 1)].wait()
    pend_s[2 * (_RNCH - 1) + 1].wait()
    if _RNCH >= 2:
        pend_s[2 * (_RNCH - 2)].wait()
        pend_s[2 * (_RNCH - 2) + 1].wait()


def _sc_reorder(x2d, idx0_3d, idx1_3d):
    ncol = x2d.shape[1]
    k = pl.kernel(
        _sc_reorder_body,
        mesh=plsc.VectorSubcoreMesh(core_axis_name="c", subcore_axis_name="s"),
        out_type=jax.ShapeDtypeStruct((S_PAD, ncol), x2d.dtype),
        scratch_types=[pltpu.VMEM((_RNCH, _RCH), jnp.int32),
                       pltpu.VMEM((_RNCH, _RCH), jnp.int32),
                       pltpu.VMEM((_RCH, ncol), x2d.dtype),
                       pltpu.VMEM((_RCH, ncol), x2d.dtype),
                       pltpu.SemaphoreType.DMA,
                       pltpu.SemaphoreType.DMA,
                       pltpu.SemaphoreType.DMA,
                       pltpu.SemaphoreType.DMA],
    )
    return k(x2d, idx0_3d, idx1_3d)


_GCH = 32


def _make_sc_gather_body(nr):
    rpw = nr // NW
    nch = rpw // _GCH

    def body(src_hbm, idx_hbm, out_hbm, idx_v, buf0, buf1, gs0, gs1, ws0, ws1):
        wid = lax.axis_index("s") * NC + lax.axis_index("c")
        base = wid * rpw
        pltpu.sync_copy(idx_hbm.at[pl.ds(base, rpw)], idx_v)
        bufs = (buf0, buf1)
        gsems = (gs0, gs1)
        wsems = (ws0, ws1)
        pend_g = {}
        pend_w = {}

        def start_gather(c):
            pend_g[c] = pltpu.async_copy(
                src_hbm.at[idx_v.at[pl.ds(c * _GCH, _GCH)]],
                bufs[c % 2], gsems[c % 2])

        start_gather(0)
        for c in range(nch):
            if c + 1 < nch:
                if c >= 1:
                    pend_w[c - 1].wait()
                start_gather(c + 1)
            pend_g[c].wait()
            pend_w[c] = pltpu.async_copy(
                bufs[c % 2], out_hbm.at[pl.ds(base + c * _GCH, _GCH)],
                wsems[c % 2])
        pend_w[nch - 1].wait()
        if nch >= 2:
            pend_w[nch - 2].wait()

    return body, rpw


def _sc_gather(src, idx, nr):
    body, rpw = _make_sc_gather_body(nr)
    k = pl.kernel(
        body,
        mesh=plsc.VectorSubcoreMesh(core_axis_name="c", subcore_axis_name="s"),
        out_type=jax.ShapeDtypeStruct((nr, D), src.dtype),
        scratch_types=[pltpu.VMEM((rpw,), jnp.int32),
                       pltpu.VMEM((_GCH, D), src.dtype),
                       pltpu.VMEM((_GCH, D), src.dtype),
                       pltpu.SemaphoreType.DMA,
                       pltpu.SemaphoreType.DMA,
                       pltpu.SemaphoreType.DMA,
                       pltpu.SemaphoreType.DMA],
    )
    return k(src, idx)


def _mm_kernel(be_ref, xs_ref, w_ref, eb_ref, os_ref):
    y = jnp.dot(xs_ref[...], w_ref[0], preferred_element_type=jnp.float32)
    os_ref[...] = y + eb_ref[0]


def _grouped_mm(xs, expert_w, eb3d, block_expert):
    grid_spec = pltpu.PrefetchScalarGridSpec(
        num_scalar_prefetch=1,
        grid=(NB,),
        in_specs=[
            pl.BlockSpec((BLK, D), lambda b, be: (b, 0)),
            pl.BlockSpec((1, D, D), lambda b, be: (be[b], 0, 0)),
            pl.BlockSpec((1, 1, D), lambda b, be: (be[b], 0, 0)),
        ],
        out_specs=pl.BlockSpec((BLK, D), lambda b, be: (b, 0)),
    )
    return pl.pallas_call(
        _mm_kernel,
        grid_spec=grid_spec,
        out_shape=jax.ShapeDtypeStruct((S_PAD, D), jnp.float32),
    )(block_expert, xs, expert_w, eb3d)


def _wadd_kernel(a_ref, b_ref, v_ref, o_ref):
    w = v_ref[...]
    o_ref[...] = a_ref[...] * w[:, 0:1] + b_ref[...] * w[:, 1:2]


def _tc_wadd(gcat, top2v):
    return pl.pallas_call(
        _wadd_kernel,
        grid=(NT,),
        in_specs=[pl.BlockSpec((TBLK, D), lambda t: (t, 0)),
                  pl.BlockSpec((TBLK, D), lambda t: (t + NT, 0)),
                  pl.BlockSpec((TBLK, K), lambda t: (t, 0))],
        out_specs=pl.BlockSpec((TBLK, D), lambda t: (t, 0)),
        out_shape=jax.ShapeDtypeStruct((N, D), jnp.float32),
    )(gcat, gcat, top2v)


@jax.jit
def kernel(hidden_states, gate_w, gate_b, expert_w, expert_b):
    x2d = hidden_states.reshape(N, D)
    eb3d = expert_b.reshape(E, 1, D)

    logits = (hidden_states @ gate_w + gate_b).reshape(N, E)

    top2v, pos, counts = _gate(logits)

    counts_pad = ((counts[0] + BLK - 1) // BLK) * BLK
    ends = jnp.cumsum(counts_pad)
    block_expert = jnp.minimum(
        jnp.sum((jnp.arange(NB, dtype=jnp.int32)[:, None] * BLK
                 >= ends[None, :]).astype(jnp.int32), 1),
        E - 1).astype(jnp.int32)
    idx0_3d = pos[:, 0].reshape(NW, _RNCH, _RCH)
    idx1_3d = pos[:, 1].reshape(NW, _RNCH, _RCH)
    idxcat = jnp.concatenate([pos[:, 0], pos[:, 1]]).astype(jnp.int32)

    xs = _sc_reorder(x2d, idx0_3d, idx1_3d)

    os_ = _grouped_mm(xs, expert_w, eb3d, block_expert)

    gcat = _sc_gather(os_, idxcat, 2 * N)
    out = _tc_wadd(gcat, top2v)

    return out.reshape(B, T, D)


# speedup vs baseline: 2.7345x; 1.0741x over previous
import jax
import jax.numpy as jnp
from jax import lax
from jax.experimental import pallas as pl
from jax.experimental.pallas import tpu as pltpu
from jax.experimental.pallas import tpu_sc as plsc

B = 2
T = 2048
D = 1024
E = 8
K = 2
N = B * T
S = N * K

TBLK = 512
NT = N // TBLK

BLK = 256
S_PAD = S + E * BLK
NB = S_PAD // BLK

NC = 2
NS = 16
NW = NC * NS


def _gate_kernel(logits_ref, v_ref, pos_ref, cnt_ref, p_s, carry_s, tril_s):
    ph = pl.program_id(0)
    t = pl.program_id(1)

    logits = logits_ref[...]
    iota = lax.broadcasted_iota(jnp.int32, (TBLK, E), 1)
    l1 = jnp.max(logits, axis=-1, keepdims=True)
    i1 = jnp.min(jnp.where(logits == l1, iota, E), axis=-1, keepdims=True)
    lm = jnp.where(iota == i1, -jnp.inf, logits)
    l2 = jnp.max(lm, axis=-1, keepdims=True)
    i2 = jnp.min(jnp.where(lm == l2, iota, E), axis=-1, keepdims=True)

    ohsum = ((iota == i1) | (iota == i2)).astype(jnp.float32)

    @pl.when(ph == 0)
    def _phase_count():
        @pl.when(t == 0)
        def _init():
            carry_s[...] = jnp.zeros_like(carry_s)
            r_iota = lax.broadcasted_iota(jnp.int32, (TBLK, TBLK), 0)
            c_iota = lax.broadcasted_iota(jnp.int32, (TBLK, TBLK), 1)
            tril_s[...] = (c_iota < r_iota).astype(jnp.float32)

        pref = jnp.dot(tril_s[...], ohsum, preferred_element_type=jnp.float32)
        p_s[pl.ds(t * TBLK, TBLK), :] = pref + carry_s[...]
        carry_s[...] = carry_s[...] + jnp.sum(ohsum, axis=0, keepdims=True)

    @pl.when(ph == 1)
    def _phase_pos():
        pexp = jnp.exp(logits - l1)
        probs = pexp / jnp.sum(pexp, axis=-1, keepdims=True)
        m1 = jnp.sum(jnp.where(iota == i1, probs, 0.0), axis=-1, keepdims=True)
        m2 = jnp.sum(jnp.where(iota == i2, probs, 0.0), axis=-1, keepdims=True)
        v_ref[...] = jnp.concatenate([m1, m2], axis=-1)
        counts = carry_s[...]
        counts_pad = jnp.ceil(counts * (1.0 / BLK)) * BLK
        e_r = lax.broadcasted_iota(jnp.int32, (E, E), 0)
        e_c = lax.broadcasted_iota(jnp.int32, (E, E), 1)
        upper = (e_r < e_c).astype(jnp.float32)
        gstart = jnp.dot(counts_pad, upper,
                         preferred_element_type=jnp.float32,
                         precision=lax.Precision.HIGHEST)
        dest = gstart + p_s[pl.ds(t * TBLK, TBLK), :]
        pos0 = jnp.sum(jnp.where(iota == i1, dest, 0.0), axis=-1,
                       keepdims=True)
        pos1 = jnp.sum(jnp.where(iota == i2, dest, 0.0), axis=-1,
                       keepdims=True)
        pos_ref[...] = jnp.concatenate([pos0, pos1], axis=-1).astype(jnp.int32)
        cnt_ref[...] = counts.astype(jnp.int32)


def _gate(logits):
    return pl.pallas_call(
        _gate_kernel,
        grid=(2, NT),
        in_specs=[pl.BlockSpec((TBLK, E), lambda p, t: (t, 0))],
        out_specs=[pl.BlockSpec((TBLK, K), lambda p, t: (t, 0)),
                   pl.BlockSpec((TBLK, K), lambda p, t: (t, 0)),
                   pl.BlockSpec((1, E), lambda p, t: (0, 0))],
        out_shape=[jax.ShapeDtypeStruct((N, K), jnp.float32),
                   jax.ShapeDtypeStruct((N, K), jnp.int32),
                   jax.ShapeDtypeStruct((1, E), jnp.int32)],
        scratch_shapes=[pltpu.VMEM((N, E), jnp.float32),
                        pltpu.VMEM((1, E), jnp.float32),
                        pltpu.VMEM((TBLK, TBLK), jnp.float32)],
    )(logits)


_RCH = 32
_RPW = N // NW
_RNCH = _RPW // _RCH


def _sc_reorder_body(x_hbm, idx0_hbm, idx1_hbm, out_hbm,
                     i0_v, i1_v, buf0, buf1, ls0, ls1, ss0, ss1):
    wid = lax.axis_index("s") * NC + lax.axis_index("c")
    base = wid * _RPW
    pltpu.sync_copy(idx0_hbm.at[wid], i0_v)
    pltpu.sync_copy(idx1_hbm.at[wid], i1_v)
    bufs = (buf0, buf1)
    lsems = (ls0, ls1)
    ssems = (ss0, ss1)
    pend_l = {}
    pend_s = {}

    def start_load(c):
        pend_l[c] = pltpu.async_copy(
            x_hbm.at[pl.ds(base + c * _RCH, _RCH)], bufs[c % 2],
            lsems[c % 2])

    start_load(0)
    for c in range(_RNCH):
        if c + 1 < _RNCH:
            if c >= 1:
                pend_s[2 * (c - 1)].wait()
                pend_s[2 * (c - 1) + 1].wait()
            start_load(c + 1)
        pend_l[c].wait()
        pend_s[2 * c] = pltpu.async_copy(
            bufs[c % 2], out_hbm.at[i0_v.at[c]], ssems[c % 2])
        pend_s[2 * c + 1] = pltpu.async_copy(
            bufs[c % 2], out_hbm.at[i1_v.at[c]], ssems[c % 2])
    pend_s[2 * (_RNCH - 1)].wait()
    pend_s[2 * (_RNCH - 1) + 1].wait()
    if _RNCH >= 2:
        pend_s[2 * (_RNCH - 2)].wait()
        pend_s[2 * (_RNCH - 2) + 1].wait()


def _sc_reorder(x2d, idx0_3d, idx1_3d):
    ncol = x2d.shape[1]
    k = pl.kernel(
        _sc_reorder_body,
        mesh=plsc.VectorSubcoreMesh(core_axis_name="c", subcore_axis_name="s"),
        out_type=jax.ShapeDtypeStruct((S_PAD, ncol), x2d.dtype),
        scratch_types=[pltpu.VMEM((_RNCH, _RCH), jnp.int32),
                       pltpu.VMEM((_RNCH, _RCH), jnp.int32),
                       pltpu.VMEM((_RCH, ncol), x2d.dtype),
                       pltpu.VMEM((_RCH, ncol), x2d.dtype),
                       pltpu.SemaphoreType.DMA,
                       pltpu.SemaphoreType.DMA,
                       pltpu.SemaphoreType.DMA,
                       pltpu.SemaphoreType.DMA],
    )
    return k(x2d, idx0_3d, idx1_3d)


_GCH = 32


def _make_sc_gather_body(nr):
    rpw = nr // NW
    nch = rpw // _GCH

    def body(src_hbm, idx_hbm, out_hbm, idx_v, buf0, buf1, gs0, gs1, ws0, ws1):
        wid = lax.axis_index("s") * NC + lax.axis_index("c")
        base = wid * rpw
        pltpu.sync_copy(idx_hbm.at[pl.ds(base, rpw)], idx_v)
        bufs = (buf0, buf1)
        gsems = (gs0, gs1)
        wsems = (ws0, ws1)
        pend_g = {}
        pend_w = {}

        def start_gather(c):
            pend_g[c] = pltpu.async_copy(
                src_hbm.at[idx_v.at[pl.ds(c * _GCH, _GCH)]],
                bufs[c % 2], gsems[c % 2])

        start_gather(0)
        for c in range(nch):
            if c + 1 < nch:
                if c >= 1:
                    pend_w[c - 1].wait()
                start_gather(c + 1)
            pend_g[c].wait()
            pend_w[c] = pltpu.async_copy(
                bufs[c % 2], out_hbm.at[pl.ds(base + c * _GCH, _GCH)],
                wsems[c % 2])
        pend_w[nch - 1].wait()
        if nch >= 2:
            pend_w[nch - 2].wait()

    return body, rpw


def _sc_gather(src, idx, nr):
    body, rpw = _make_sc_gather_body(nr)
    k = pl.kernel(
        body,
        mesh=plsc.VectorSubcoreMesh(core_axis_name="c", subcore_axis_name="s"),
        out_type=jax.ShapeDtypeStruct((nr, D), src.dtype),
        scratch_types=[pltpu.VMEM((rpw,), jnp.int32),
                       pltpu.VMEM((_GCH, D), src.dtype),
                       pltpu.VMEM((_GCH, D), src.dtype),
                       pltpu.SemaphoreType.DMA,
                       pltpu.SemaphoreType.DMA,
                       pltpu.SemaphoreType.DMA,
                       pltpu.SemaphoreType.DMA],
    )
    return k(src, idx)


def _mm_kernel(be_ref, xs_ref, w_ref, eb_ref, os_ref):
    y = jnp.dot(xs_ref[...], w_ref[0], preferred_element_type=jnp.float32)
    os_ref[...] = y + eb_ref[0]


def _grouped_mm(xs, expert_w, eb3d, block_expert):
    grid_spec = pltpu.PrefetchScalarGridSpec(
        num_scalar_prefetch=1,
        grid=(NB,),
        in_specs=[
            pl.BlockSpec((BLK, D), lambda b, be: (b, 0)),
            pl.BlockSpec((1, D, D), lambda b, be: (be[b], 0, 0)),
            pl.BlockSpec((1, 1, D), lambda b, be: (be[b], 0, 0)),
        ],
        out_specs=pl.BlockSpec((BLK, D), lambda b, be: (b, 0)),
    )
    return pl.pallas_call(
        _mm_kernel,
        grid_spec=grid_spec,
        out_shape=jax.ShapeDtypeStruct((S_PAD, D), jnp.float32),
    )(block_expert, xs, expert_w, eb3d)


def _wadd_kernel(a_ref, b_ref, v_ref, o_ref):
    w = v_ref[...]
    o_ref[...] = a_ref[...] * w[:, 0:1] + b_ref[...] * w[:, 1:2]


def _tc_wadd(gcat, top2v):
    return pl.pallas_call(
        _wadd_kernel,
        grid=(NT,),
        in_specs=[pl.BlockSpec((TBLK, D), lambda t: (t, 0)),
                  pl.BlockSpec((TBLK, D), lambda t: (t + NT, 0)),
                  pl.BlockSpec((TBLK, K), lambda t: (t, 0))],
        out_specs=pl.BlockSpec((TBLK, D), lambda t: (t, 0)),
        out_shape=jax.ShapeDtypeStruct((N, D), jnp.float32),
    )(gcat, gcat, top2v)


@jax.jit
def kernel(hidden_states, gate_w, gate_b, expert_w, expert_b):
    x2d = hidden_states.reshape(N, D)
    eb3d = expert_b.reshape(E, 1, D)

    logits = (hidden_states @ gate_w + gate_b).reshape(N, E)

    top2v, pos, counts = _gate(logits)

    counts_pad = ((counts[0] + BLK - 1) // BLK) * BLK
    ends = jnp.cumsum(counts_pad)
    block_expert = jnp.minimum(
        jnp.sum((jnp.arange(NB, dtype=jnp.int32)[:, None] * BLK
                 >= ends[None, :]).astype(jnp.int32), 1),
        E - 1).astype(jnp.int32)
    idx0_3d = pos[:, 0].reshape(NW, _RNCH, _RCH)
    idx1_3d = pos[:, 1].reshape(NW, _RNCH, _RCH)
    idxcat = jnp.concatenate([pos[:, 0], pos[:, 1]]).astype(jnp.int32)

    xs = _sc_reorder(x2d, idx0_3d, idx1_3d)

    os_ = _grouped_mm(xs, expert_w, eb3d, block_expert)

    gcat = _sc_gather(os_, idxcat, 2 * N)
    out = _tc_wadd(gcat, top2v)

    return out.reshape(B, T, D)
